# initial kernel scaffold (unmeasured)
import jax
import jax.numpy as jnp
from jax import lax
from jax.experimental import pallas as pl
from jax.experimental.pallas import tpu as pltpu

T_LOC = 1024
T_GLB = 2048
D = 1024
F = 4096
E = 16
E_LOC = 8
K = 2
CAP = 384
FB = 1024


def _peer():
    ix = lax.axis_index("x")
    iy = lax.axis_index("y")
    iz = lax.axis_index("z")
    return ix, (1 - ix, iy, iz)


def _xpeer_barrier(peer):
    barrier = pltpu.get_barrier_semaphore()
    pl.semaphore_signal(
        barrier, inc=1, device_id=peer, device_id_type=pl.DeviceIdType.MESH
    )
    pl.semaphore_wait(barrier, 1)


def _ag_body(x_ref, rme_ref, xall_ref, gall_ref, rpeer_ref, send_sems, recv_sems):
    ix, peer = _peer()
    _xpeer_barrier(peer)

    my_off = ix * T_LOC

    r_rdma = pltpu.make_async_remote_copy(
        src_ref=rme_ref,
        dst_ref=rpeer_ref,
        send_sem=send_sems.at[0],
        recv_sem=recv_sems.at[0],
        device_id=peer,
        device_id_type=pl.DeviceIdType.MESH,
    )
    r_rdma.start()

    xall_ref[pl.ds(my_off, T_LOC), :] = x_ref[...].astype(jnp.bfloat16)
    x_rdma = pltpu.make_async_remote_copy(
        src_ref=xall_ref.at[pl.ds(my_off, T_LOC), :],
        dst_ref=xall_ref.at[pl.ds(my_off, T_LOC), :],
        send_sem=send_sems.at[1],
        recv_sem=recv_sems.at[1],
        device_id=peer,
        device_id_type=pl.DeviceIdType.MESH,
    )
    x_rdma.start()

    gme = jnp.dot(x_ref[...], rme_ref[...], preferred_element_type=jnp.float32)
    r_rdma.wait_recv()
    gpe = jnp.dot(x_ref[...], rpeer_ref[...], preferred_element_type=jnp.float32)
    cols = jnp.where(
        ix == 0,
        jnp.concatenate([gme, gpe], axis=1),
        jnp.concatenate([gpe, gme], axis=1),
    )
    gall_ref[pl.ds(my_off, T_LOC), :] = cols
    g_rdma = pltpu.make_async_remote_copy(
        src_ref=gall_ref.at[pl.ds(my_off, T_LOC), :],
        dst_ref=gall_ref.at[pl.ds(my_off, T_LOC), :],
        send_sem=send_sems.at[2],
        recv_sem=recv_sems.at[2],
        device_id=peer,
        device_id_type=pl.DeviceIdType.MESH,
    )
    g_rdma.start()

    r_rdma.wait_send()
    x_rdma.wait()
    g_rdma.wait()


def _ag_call(x, router):
    return pl.pallas_call(
        _ag_body,
        out_shape=[
            jax.ShapeDtypeStruct((T_GLB, D), jnp.bfloat16),
            jax.ShapeDtypeStruct((T_GLB, E), jnp.float32),
        ],
        in_specs=[
            pl.BlockSpec(memory_space=pltpu.VMEM),
            pl.BlockSpec(memory_space=pltpu.VMEM),
        ],
        out_specs=[
            pl.BlockSpec(memory_space=pltpu.VMEM),
            pl.BlockSpec(memory_space=pltpu.VMEM),
        ],
        scratch_shapes=[
            pltpu.VMEM((T_LOC, E_LOC), jnp.float32),
            pltpu.SemaphoreType.DMA((3,)),
            pltpu.SemaphoreType.DMA((3,)),
        ],
        compiler_params=pltpu.CompilerParams(collective_id=0),
    )(x, router)


def _ffn_body(xg_ref, w1_ref, w2_ref, wg_ref, out_ref):
    f = pl.program_id(1)
    nf = pl.num_programs(1)
    xg = xg_ref[0]
    h = jnp.maximum(
        jnp.dot(xg, w1_ref[0].astype(jnp.bfloat16), preferred_element_type=jnp.float32),
        0.0,
    )
    y = jnp.dot(
        h.astype(jnp.bfloat16),
        w2_ref[0].astype(jnp.bfloat16),
        preferred_element_type=jnp.float32,
    )

    @pl.when(f == 0)
    def _():
        out_ref[0] = y

    @pl.when(f != 0)
    def _():
        out_ref[0] += y

    @pl.when(f == nf - 1)
    def _():
        out_ref[0] = out_ref[0] * wg_ref[0]


def _ffn_call(xg, W1, W2, wg):
    return pl.pallas_call(
        _ffn_body,
        grid=(E_LOC, F // FB),
        out_shape=jax.ShapeDtypeStruct((E_LOC, CAP, D), jnp.float32),
        in_specs=[
            pl.BlockSpec((1, CAP, D), lambda e, f: (e, 0, 0)),
            pl.BlockSpec((1, D, FB), lambda e, f: (e, 0, f)),
            pl.BlockSpec((1, FB, D), lambda e, f: (e, f, 0)),
            pl.BlockSpec((1, CAP, 1), lambda e, f: (e, 0, 0)),
        ],
        out_specs=pl.BlockSpec((1, CAP, D), lambda e, f: (e, 0, 0)),
        compiler_params=pltpu.CompilerParams(
            dimension_semantics=("parallel", "arbitrary"),
        ),
    )(xg, W1, W2, wg)


def _rs_body(p_ref, out_ref, sbuf_ref, rbuf_ref, send_sem, recv_sem):
    ix, peer = _peer()
    _xpeer_barrier(peer)

    sbuf_ref[...] = p_ref[pl.ds((1 - ix) * T_LOC, T_LOC), :].astype(jnp.bfloat16)
    rdma = pltpu.make_async_remote_copy(
        src_ref=sbuf_ref,
        dst_ref=rbuf_ref,
        send_sem=send_sem,
        recv_sem=recv_sem,
        device_id=peer,
        device_id_type=pl.DeviceIdType.MESH,
    )
    rdma.start()
    out_ref[...] = p_ref[pl.ds(ix * T_LOC, T_LOC), :]
    rdma.wait()
    out_ref[...] += rbuf_ref[...].astype(jnp.float32)


def _rs_call(partial):
    return pl.pallas_call(
        _rs_body,
        out_shape=jax.ShapeDtypeStruct((T_LOC, D), jnp.float32),
        in_specs=[pl.BlockSpec(memory_space=pltpu.VMEM)],
        out_specs=pl.BlockSpec(memory_space=pltpu.VMEM),
        scratch_shapes=[
            pltpu.VMEM((T_LOC, D), jnp.bfloat16),
            pltpu.VMEM((T_LOC, D), jnp.bfloat16),
            pltpu.SemaphoreType.DMA,
            pltpu.SemaphoreType.DMA,
        ],
        compiler_params=pltpu.CompilerParams(collective_id=1),
    )(partial)


def kernel(x, router, W1, W2):
    ix = lax.axis_index("x")

    x_all, gates = _ag_call(x, router)

    tv, ti = lax.top_k(gates, K)
    w = jax.nn.softmax(tv, axis=1)
    flat_e = ti.reshape(-1)
    flat_t = jnp.arange(T_GLB * K, dtype=jnp.int32) // K
    flat_w = w.reshape(-1)

    le = flat_e - E_LOC * ix
    local = (le >= 0) & (le < E_LOC)
    oh = (le[:, None] == jnp.arange(E_LOC)[None, :]) & local[:, None]
    pos = jnp.cumsum(oh.astype(jnp.int32), axis=0)
    rank = jnp.sum(jnp.where(oh, pos - 1, 0), axis=1)
    slot = jnp.where(local & (rank < CAP), le * CAP + rank, E_LOC * CAP)

    buf_tok = jnp.zeros(E_LOC * CAP + 1, jnp.int32).at[slot].set(flat_t)[: E_LOC * CAP]
    buf_w = jnp.zeros(E_LOC * CAP + 1, jnp.float32).at[slot].set(flat_w)[: E_LOC * CAP]

    xg = x_all[buf_tok].reshape(E_LOC, CAP, D)
    wg = buf_w.reshape(E_LOC, CAP, 1)

    yg = _ffn_call(xg, W1, W2, wg)

    partial = (
        jnp.zeros((T_GLB, D), jnp.float32).at[buf_tok].add(yg.reshape(E_LOC * CAP, D))
    )
    return _rs_call(partial)


# baseline (device time: 247683 ns/iter reference)
import jax
import jax.numpy as jnp
from jax import lax
from jax.experimental import pallas as pl
from jax.experimental.pallas import tpu as pltpu

T_LOC = 1024
T_GLB = 2048
D = 1024
F = 4096
E = 16
E_LOC = 8
K = 2
CAP = 384
FB = 1024


def _peer():
    ix = lax.axis_index("x")
    iy = lax.axis_index("y")
    iz = lax.axis_index("z")
    return ix, (1 - ix, iy, iz)


def _xpeer_barrier(peer):
    barrier = pltpu.get_barrier_semaphore()
    pl.semaphore_signal(
        barrier, inc=1, device_id=peer, device_id_type=pl.DeviceIdType.MESH
    )
    pl.semaphore_wait(barrier, 1)


def _ag_body(x_ref, rme_ref, xall_ref, gall_ref, rpeer_ref, send_sems, recv_sems):
    ix, peer = _peer()
    _xpeer_barrier(peer)

    my_off = ix * T_LOC

    r_rdma = pltpu.make_async_remote_copy(
        src_ref=rme_ref,
        dst_ref=rpeer_ref,
        send_sem=send_sems.at[0],
        recv_sem=recv_sems.at[0],
        device_id=peer,
        device_id_type=pl.DeviceIdType.MESH,
    )
    r_rdma.start()

    xall_ref[pl.ds(my_off, T_LOC), :] = x_ref[...].astype(jnp.bfloat16)
    x_rdma = pltpu.make_async_remote_copy(
        src_ref=xall_ref.at[pl.ds(my_off, T_LOC), :],
        dst_ref=xall_ref.at[pl.ds(my_off, T_LOC), :],
        send_sem=send_sems.at[1],
        recv_sem=recv_sems.at[1],
        device_id=peer,
        device_id_type=pl.DeviceIdType.MESH,
    )
    x_rdma.start()

    gme = jnp.dot(
        x_ref[...],
        rme_ref[...],
        preferred_element_type=jnp.float32,
        precision=lax.Precision.HIGHEST,
    )
    r_rdma.wait_recv()
    gpe = jnp.dot(
        x_ref[...],
        rpeer_ref[...],
        preferred_element_type=jnp.float32,
        precision=lax.Precision.HIGHEST,
    )
    cols = jnp.where(
        ix == 0,
        jnp.concatenate([gme, gpe], axis=1),
        jnp.concatenate([gpe, gme], axis=1),
    )
    gall_ref[pl.ds(my_off, T_LOC), :] = cols
    g_rdma = pltpu.make_async_remote_copy(
        src_ref=gall_ref.at[pl.ds(my_off, T_LOC), :],
        dst_ref=gall_ref.at[pl.ds(my_off, T_LOC), :],
        send_sem=send_sems.at[2],
        recv_sem=recv_sems.at[2],
        device_id=peer,
        device_id_type=pl.DeviceIdType.MESH,
    )
    g_rdma.start()

    r_rdma.wait_send()
    x_rdma.wait()
    g_rdma.wait()


def _ag_call(x, router):
    return pl.pallas_call(
        _ag_body,
        out_shape=[
            jax.ShapeDtypeStruct((T_GLB, D), jnp.bfloat16),
            jax.ShapeDtypeStruct((T_GLB, E), jnp.float32),
        ],
        in_specs=[
            pl.BlockSpec(memory_space=pltpu.VMEM),
            pl.BlockSpec(memory_space=pltpu.VMEM),
        ],
        out_specs=[
            pl.BlockSpec(memory_space=pltpu.VMEM),
            pl.BlockSpec(memory_space=pltpu.VMEM),
        ],
        scratch_shapes=[
            pltpu.VMEM((T_LOC, E_LOC), jnp.float32),
            pltpu.SemaphoreType.DMA((3,)),
            pltpu.SemaphoreType.DMA((3,)),
        ],
        compiler_params=pltpu.CompilerParams(collective_id=0),
    )(x, router)


def _ffn_body(xg_ref, w1_ref, w2_ref, wg_ref, out_ref):
    f = pl.program_id(1)
    nf = pl.num_programs(1)
    xg = xg_ref[0]
    h = jnp.maximum(
        jnp.dot(xg, w1_ref[0].astype(jnp.bfloat16), preferred_element_type=jnp.float32),
        0.0,
    )
    y = jnp.dot(
        h.astype(jnp.bfloat16),
        w2_ref[0].astype(jnp.bfloat16),
        preferred_element_type=jnp.float32,
    )

    @pl.when(f == 0)
    def _():
        out_ref[0] = y

    @pl.when(f != 0)
    def _():
        out_ref[0] += y

    @pl.when(f == nf - 1)
    def _():
        out_ref[0] = out_ref[0] * wg_ref[0]


def _ffn_call(xg, W1, W2, wg):
    return pl.pallas_call(
        _ffn_body,
        grid=(E_LOC, F // FB),
        out_shape=jax.ShapeDtypeStruct((E_LOC, CAP, D), jnp.float32),
        in_specs=[
            pl.BlockSpec((1, CAP, D), lambda e, f: (e, 0, 0)),
            pl.BlockSpec((1, D, FB), lambda e, f: (e, 0, f)),
            pl.BlockSpec((1, FB, D), lambda e, f: (e, f, 0)),
            pl.BlockSpec((1, CAP, 1), lambda e, f: (e, 0, 0)),
        ],
        out_specs=pl.BlockSpec((1, CAP, D), lambda e, f: (e, 0, 0)),
        compiler_params=pltpu.CompilerParams(
            dimension_semantics=("parallel", "arbitrary"),
        ),
    )(xg, W1, W2, wg)


def _rs_body(p_ref, out_ref, sbuf_ref, rbuf_ref, send_sem, recv_sem):
    ix, peer = _peer()
    _xpeer_barrier(peer)

    sbuf_ref[...] = p_ref[pl.ds((1 - ix) * T_LOC, T_LOC), :].astype(jnp.bfloat16)
    rdma = pltpu.make_async_remote_copy(
        src_ref=sbuf_ref,
        dst_ref=rbuf_ref,
        send_sem=send_sem,
        recv_sem=recv_sem,
        device_id=peer,
        device_id_type=pl.DeviceIdType.MESH,
    )
    rdma.start()
    out_ref[...] = p_ref[pl.ds(ix * T_LOC, T_LOC), :]
    rdma.wait()
    out_ref[...] += rbuf_ref[...].astype(jnp.float32)


def _rs_call(partial):
    return pl.pallas_call(
        _rs_body,
        out_shape=jax.ShapeDtypeStruct((T_LOC, D), jnp.float32),
        in_specs=[pl.BlockSpec(memory_space=pltpu.VMEM)],
        out_specs=pl.BlockSpec(memory_space=pltpu.VMEM),
        scratch_shapes=[
            pltpu.VMEM((T_LOC, D), jnp.bfloat16),
            pltpu.VMEM((T_LOC, D), jnp.bfloat16),
            pltpu.SemaphoreType.DMA,
            pltpu.SemaphoreType.DMA,
        ],
        compiler_params=pltpu.CompilerParams(collective_id=1),
    )(partial)


def kernel(x, router, W1, W2):
    ix = lax.axis_index("x")

    x_all, gates = _ag_call(x, router)

    tv, ti = lax.top_k(gates, K)
    w = jax.nn.softmax(tv, axis=1)
    flat_e = ti.reshape(-1)
    flat_t = jnp.arange(T_GLB * K, dtype=jnp.int32) // K
    flat_w = w.reshape(-1)

    le = flat_e - E_LOC * ix
    local = (le >= 0) & (le < E_LOC)
    oh = (le[:, None] == jnp.arange(E_LOC)[None, :]) & local[:, None]
    pos = jnp.cumsum(oh.astype(jnp.int32), axis=0)
    rank = jnp.sum(jnp.where(oh, pos - 1, 0), axis=1)
    slot = jnp.where(local & (rank < CAP), le * CAP + rank, E_LOC * CAP)

    buf_tok = jnp.zeros(E_LOC * CAP + 1, jnp.int32).at[slot].set(flat_t)[: E_LOC * CAP]
    buf_w = jnp.zeros(E_LOC * CAP + 1, jnp.float32).at[slot].set(flat_w)[: E_LOC * CAP]

    xg = x_all[buf_tok].reshape(E_LOC, CAP, D)
    wg = buf_w.reshape(E_LOC, CAP, 1)

    yg = _ffn_call(xg, W1, W2, wg)

    partial = (
        jnp.zeros((T_GLB, D), jnp.float32).at[buf_tok].add(yg.reshape(E_LOC * CAP, D))
    )
    return _rs_call(partial)


# device time: 234412 ns/iter; 1.0566x vs baseline; 1.0566x over previous
import jax
import jax.numpy as jnp
from jax import lax
from jax.experimental import pallas as pl
from jax.experimental.pallas import tpu as pltpu

T_LOC = 1024
T_GLB = 2048
D = 1024
F = 4096
E = 16
E_LOC = 8
K = 2
NPAIR = T_GLB * K
CAP = 320
FB = 1024


def _peer():
    ix = lax.axis_index("x")
    iy = lax.axis_index("y")
    iz = lax.axis_index("z")
    return ix, (1 - ix, iy, iz)


def _xpeer_barrier(peer):
    barrier = pltpu.get_barrier_semaphore()
    pl.semaphore_signal(
        barrier, inc=1, device_id=peer, device_id_type=pl.DeviceIdType.MESH
    )
    pl.semaphore_wait(barrier, 1)


def _ag_body(x_ref, rme_ref, xall_ref, gall_ref, rpeer_ref, send_sems, recv_sems):
    ix, peer = _peer()
    _xpeer_barrier(peer)

    my_off = ix * T_LOC

    r_rdma = pltpu.make_async_remote_copy(
        src_ref=rme_ref,
        dst_ref=rpeer_ref,
        send_sem=send_sems.at[0],
        recv_sem=recv_sems.at[0],
        device_id=peer,
        device_id_type=pl.DeviceIdType.MESH,
    )
    r_rdma.start()

    xall_ref[pl.ds(my_off, T_LOC), :] = x_ref[...].astype(jnp.bfloat16)
    x_rdma = pltpu.make_async_remote_copy(
        src_ref=xall_ref.at[pl.ds(my_off, T_LOC), :],
        dst_ref=xall_ref.at[pl.ds(my_off, T_LOC), :],
        send_sem=send_sems.at[1],
        recv_sem=recv_sems.at[1],
        device_id=peer,
        device_id_type=pl.DeviceIdType.MESH,
    )
    x_rdma.start()

    gme = jnp.dot(
        x_ref[...],
        rme_ref[...],
        preferred_element_type=jnp.float32,
        precision=lax.Precision.HIGHEST,
    )
    r_rdma.wait_recv()
    gpe = jnp.dot(
        x_ref[...],
        rpeer_ref[...],
        preferred_element_type=jnp.float32,
        precision=lax.Precision.HIGHEST,
    )
    cols = jnp.where(
        ix == 0,
        jnp.concatenate([gme, gpe], axis=1),
        jnp.concatenate([gpe, gme], axis=1),
    )
    gall_ref[pl.ds(my_off, T_LOC), :] = cols
    g_rdma = pltpu.make_async_remote_copy(
        src_ref=gall_ref.at[pl.ds(my_off, T_LOC), :],
        dst_ref=gall_ref.at[pl.ds(my_off, T_LOC), :],
        send_sem=send_sems.at[2],
        recv_sem=recv_sems.at[2],
        device_id=peer,
        device_id_type=pl.DeviceIdType.MESH,
    )
    g_rdma.start()

    r_rdma.wait_send()
    x_rdma.wait()
    g_rdma.wait()


def _ag_call(x, router):
    return pl.pallas_call(
        _ag_body,
        out_shape=[
            jax.ShapeDtypeStruct((T_GLB, D), jnp.bfloat16),
            jax.ShapeDtypeStruct((T_GLB, E), jnp.float32),
        ],
        in_specs=[
            pl.BlockSpec(memory_space=pltpu.VMEM),
            pl.BlockSpec(memory_space=pltpu.VMEM),
        ],
        out_specs=[
            pl.BlockSpec(memory_space=pltpu.VMEM),
            pl.BlockSpec(memory_space=pltpu.VMEM),
        ],
        scratch_shapes=[
            pltpu.VMEM((T_LOC, E_LOC), jnp.float32),
            pltpu.SemaphoreType.DMA((3,)),
            pltpu.SemaphoreType.DMA((3,)),
        ],
        compiler_params=pltpu.CompilerParams(collective_id=0),
    )(x, router)


def _ffn_body(xg_ref, w1_ref, w2_ref, wg_ref, out_ref, acc_ref):
    f = pl.program_id(1)
    nf = pl.num_programs(1)
    xg = xg_ref[0]
    h = jnp.maximum(
        jnp.dot(xg, w1_ref[0].astype(jnp.bfloat16), preferred_element_type=jnp.float32),
        0.0,
    )
    y = jnp.dot(
        h.astype(jnp.bfloat16),
        w2_ref[0].astype(jnp.bfloat16),
        preferred_element_type=jnp.float32,
    )

    @pl.when(f == 0)
    def _():
        acc_ref[...] = y

    @pl.when(f != 0)
    def _():
        acc_ref[...] += y

    @pl.when(f == nf - 1)
    def _():
        out_ref[0] = (acc_ref[...] * wg_ref[0]).astype(jnp.bfloat16)


def _ffn_call(xg, W1, W2, wg):
    return pl.pallas_call(
        _ffn_body,
        grid=(E_LOC, F // FB),
        out_shape=jax.ShapeDtypeStruct((E_LOC, CAP, D), jnp.bfloat16),
        in_specs=[
            pl.BlockSpec((1, CAP, D), lambda e, f: (e, 0, 0)),
            pl.BlockSpec((1, D, FB), lambda e, f: (e, 0, f)),
            pl.BlockSpec((1, FB, D), lambda e, f: (e, f, 0)),
            pl.BlockSpec((1, CAP, 1), lambda e, f: (e, 0, 0)),
        ],
        out_specs=pl.BlockSpec((1, CAP, D), lambda e, f: (e, 0, 0)),
        scratch_shapes=[pltpu.VMEM((CAP, D), jnp.float32)],
        compiler_params=pltpu.CompilerParams(
            dimension_semantics=("parallel", "arbitrary"),
        ),
    )(xg, W1, W2, wg)


def _rs_body(p_ref, out_ref, rbuf_ref, send_sem, recv_sem):
    ix, peer = _peer()
    _xpeer_barrier(peer)

    rdma = pltpu.make_async_remote_copy(
        src_ref=p_ref.at[pl.ds((1 - ix) * T_LOC, T_LOC), :],
        dst_ref=rbuf_ref,
        send_sem=send_sem,
        recv_sem=recv_sem,
        device_id=peer,
        device_id_type=pl.DeviceIdType.MESH,
    )
    rdma.start()
    out_ref[...] = p_ref[pl.ds(ix * T_LOC, T_LOC), :].astype(jnp.float32)
    rdma.wait()
    out_ref[...] += rbuf_ref[...].astype(jnp.float32)


def _rs_call(partial):
    return pl.pallas_call(
        _rs_body,
        out_shape=jax.ShapeDtypeStruct((T_LOC, D), jnp.float32),
        in_specs=[pl.BlockSpec(memory_space=pltpu.VMEM)],
        out_specs=pl.BlockSpec(memory_space=pltpu.VMEM),
        scratch_shapes=[
            pltpu.VMEM((T_LOC, D), jnp.bfloat16),
            pltpu.SemaphoreType.DMA,
            pltpu.SemaphoreType.DMA,
        ],
        compiler_params=pltpu.CompilerParams(collective_id=1),
    )(partial)


def kernel(x, router, W1, W2):
    ix = lax.axis_index("x")

    x_all, gates = _ag_call(x, router)

    tv, ti = lax.top_k(gates, K)
    w = jax.nn.softmax(tv, axis=1)
    flat_e = ti.reshape(-1)
    flat_w = w.reshape(-1)

    le = flat_e - E_LOC * ix
    local = (le >= 0) & (le < E_LOC)
    ar = jnp.arange(NPAIR, dtype=jnp.int32)

    key = jnp.where(local, le * NPAIR + ar, E_LOC * NPAIR + ar)
    order = jnp.argsort(key)
    j_of_p = jnp.argsort(order)

    counts = jnp.sum(
        (le[:, None] == jnp.arange(E_LOC)[None, :]) & local[:, None],
        axis=0,
        dtype=jnp.int32,
    )
    start = jnp.concatenate(
        [jnp.zeros(1, jnp.int32), jnp.cumsum(counts)[:-1]]
    )

    rank = j_of_p - start[jnp.clip(le, 0, E_LOC - 1)]
    slot = jnp.where(local & (rank < CAP), le * CAP + rank, E_LOC * CAP)

    s_ar = jnp.arange(E_LOC * CAP, dtype=jnp.int32)
    s_e = s_ar // CAP
    s_r = s_ar % CAP
    s_valid = s_r < counts[s_e]
    pair_at_slot = jnp.where(
        s_valid, order[jnp.clip(start[s_e] + s_r, 0, NPAIR - 1)], NPAIR
    )
    buf_tok = jnp.where(pair_at_slot < NPAIR, pair_at_slot // K, 0)
    flat_w_pad = jnp.concatenate([flat_w, jnp.zeros(1, jnp.float32)])
    buf_w = flat_w_pad[jnp.clip(pair_at_slot, 0, NPAIR)]

    xg = x_all[buf_tok].reshape(E_LOC, CAP, D)
    wg = buf_w.reshape(E_LOC, CAP, 1)

    yg = _ffn_call(xg, W1, W2, wg)

    yg_pad = jnp.concatenate(
        [yg.reshape(E_LOC * CAP, D), jnp.zeros((1, D), jnp.bfloat16)]
    )
    sp = slot.reshape(T_GLB, K)
    partial = (
        yg_pad[sp[:, 0]].astype(jnp.float32) + yg_pad[sp[:, 1]].astype(jnp.float32)
    ).astype(jnp.bfloat16)

    return _rs_call(partial)


# device time: 196108 ns/iter; 1.2630x vs baseline; 1.1953x over previous
import jax
import jax.numpy as jnp
from jax import lax
from jax.experimental import pallas as pl
from jax.experimental.pallas import tpu as pltpu

T_LOC = 1024
T_GLB = 2048
D = 1024
F = 4096
E = 16
E_LOC = 8
K = 2
NPAIR = T_GLB * K
CAP = 320
NSLOT = E_LOC * CAP
FB = 1024


def _peer():
    ix = lax.axis_index("x")
    iy = lax.axis_index("y")
    iz = lax.axis_index("z")
    return ix, (1 - ix, iy, iz)


def _xpeer_barrier(peer):
    barrier = pltpu.get_barrier_semaphore()
    pl.semaphore_signal(
        barrier, inc=1, device_id=peer, device_id_type=pl.DeviceIdType.MESH
    )
    pl.semaphore_wait(barrier, 1)


def _ag_body(x_ref, rme_ref, xall_ref, gall_ref, rpeer_ref, send_sems, recv_sems):
    ix, peer = _peer()
    _xpeer_barrier(peer)

    my_off = ix * T_LOC

    r_rdma = pltpu.make_async_remote_copy(
        src_ref=rme_ref,
        dst_ref=rpeer_ref,
        send_sem=send_sems.at[0],
        recv_sem=recv_sems.at[0],
        device_id=peer,
        device_id_type=pl.DeviceIdType.MESH,
    )
    r_rdma.start()

    xall_ref[pl.ds(my_off, T_LOC), :] = x_ref[...].astype(jnp.bfloat16)
    x_rdma = pltpu.make_async_remote_copy(
        src_ref=xall_ref.at[pl.ds(my_off, T_LOC), :],
        dst_ref=xall_ref.at[pl.ds(my_off, T_LOC), :],
        send_sem=send_sems.at[1],
        recv_sem=recv_sems.at[1],
        device_id=peer,
        device_id_type=pl.DeviceIdType.MESH,
    )
    x_rdma.start()

    gme = jnp.dot(
        x_ref[...],
        rme_ref[...],
        preferred_element_type=jnp.float32,
        precision=lax.Precision.HIGHEST,
    )
    r_rdma.wait_recv()
    gpe = jnp.dot(
        x_ref[...],
        rpeer_ref[...],
        preferred_element_type=jnp.float32,
        precision=lax.Precision.HIGHEST,
    )
    cols = jnp.where(
        ix == 0,
        jnp.concatenate([gme, gpe], axis=1),
        jnp.concatenate([gpe, gme], axis=1),
    )
    gall_ref[pl.ds(my_off, T_LOC), :] = cols
    g_rdma = pltpu.make_async_remote_copy(
        src_ref=gall_ref.at[pl.ds(my_off, T_LOC), :],
        dst_ref=gall_ref.at[pl.ds(my_off, T_LOC), :],
        send_sem=send_sems.at[2],
        recv_sem=recv_sems.at[2],
        device_id=peer,
        device_id_type=pl.DeviceIdType.MESH,
    )
    g_rdma.start()

    r_rdma.wait_send()
    x_rdma.wait()
    g_rdma.wait()


def _ag_call(x, router):
    return pl.pallas_call(
        _ag_body,
        out_shape=[
            jax.ShapeDtypeStruct((T_GLB, D), jnp.bfloat16),
            jax.ShapeDtypeStruct((T_GLB, E), jnp.float32),
        ],
        in_specs=[
            pl.BlockSpec(memory_space=pltpu.VMEM),
            pl.BlockSpec(memory_space=pltpu.VMEM),
        ],
        out_specs=[
            pl.BlockSpec(memory_space=pltpu.VMEM),
            pl.BlockSpec(memory_space=pltpu.VMEM),
        ],
        scratch_shapes=[
            pltpu.VMEM((T_LOC, E_LOC), jnp.float32),
            pltpu.SemaphoreType.DMA((3,)),
            pltpu.SemaphoreType.DMA((3,)),
        ],
        compiler_params=pltpu.CompilerParams(collective_id=0),
    )(x, router)


def _ffn_body(
    xall_ref, s0_ref, s1_ref, w0_ref, w1r_ref, w1_ref, w2_ref, out_ref, xg_ref, acc_ref
):
    e = pl.program_id(0)
    f = pl.program_id(1)
    nf = pl.num_programs(1)

    @pl.when(f == 0)
    def _():
        slot_id = lax.broadcasted_iota(jnp.int32, (CAP, T_GLB), 0) + e * CAP
        m0 = slot_id == s0_ref[...]
        m1 = slot_id == s1_ref[...]
        ohw = jnp.where(m0, w0_ref[...], 0.0) + jnp.where(m1, w1r_ref[...], 0.0)
        xg_ref[...] = jnp.dot(
            ohw.astype(jnp.bfloat16),
            xall_ref[...],
            preferred_element_type=jnp.float32,
        ).astype(jnp.bfloat16)

    h = jnp.maximum(
        jnp.dot(
            xg_ref[...],
            w1_ref[0].astype(jnp.bfloat16),
            preferred_element_type=jnp.float32,
        ),
        0.0,
    )
    y = jnp.dot(
        h.astype(jnp.bfloat16),
        w2_ref[0].astype(jnp.bfloat16),
        preferred_element_type=jnp.float32,
    )

    @pl.when(f == 0)
    def _():
        acc_ref[...] = y

    @pl.when(f != 0)
    def _():
        acc_ref[...] += y

    @pl.when(f == nf - 1)
    def _():
        out_ref[0] = acc_ref[...].astype(jnp.bfloat16)


def _ffn_call(x_all, s0, s1, w0, w1r, W1, W2):
    return pl.pallas_call(
        _ffn_body,
        grid=(E_LOC, F // FB),
        out_shape=jax.ShapeDtypeStruct((E_LOC, CAP, D), jnp.bfloat16),
        in_specs=[
            pl.BlockSpec((T_GLB, D), lambda e, f: (0, 0)),
            pl.BlockSpec((1, T_GLB), lambda e, f: (0, 0)),
            pl.BlockSpec((1, T_GLB), lambda e, f: (0, 0)),
            pl.BlockSpec((1, T_GLB), lambda e, f: (0, 0)),
            pl.BlockSpec((1, T_GLB), lambda e, f: (0, 0)),
            pl.BlockSpec((1, D, FB), lambda e, f: (e, 0, f)),
            pl.BlockSpec((1, FB, D), lambda e, f: (e, f, 0)),
        ],
        out_specs=pl.BlockSpec((1, CAP, D), lambda e, f: (e, 0, 0)),
        scratch_shapes=[
            pltpu.VMEM((CAP, D), jnp.bfloat16),
            pltpu.VMEM((CAP, D), jnp.float32),
        ],
        compiler_params=pltpu.CompilerParams(
            dimension_semantics=("parallel", "arbitrary"),
        ),
    )(x_all, s0, s1, w0, w1r, W1, W2)


def _rs_body(yg_ref, sp_ref, out_ref, sbuf_ref, rbuf_ref, send_sem, recv_sem):
    ix, peer = _peer()
    _xpeer_barrier(peer)

    def half_part(off):
        s0 = sp_ref[pl.ds(off, T_LOC), 0:1]
        s1 = sp_ref[pl.ds(off, T_LOC), 1:2]
        slot_id = lax.broadcasted_iota(jnp.int32, (T_LOC, NSLOT), 1)
        oh = ((slot_id == s0) | (slot_id == s1)).astype(jnp.bfloat16)
        return jnp.dot(oh, yg_ref[...], preferred_element_type=jnp.float32)

    sbuf_ref[...] = half_part((1 - ix) * T_LOC).astype(jnp.bfloat16)
    rdma = pltpu.make_async_remote_copy(
        src_ref=sbuf_ref,
        dst_ref=rbuf_ref,
        send_sem=send_sem,
        recv_sem=recv_sem,
        device_id=peer,
        device_id_type=pl.DeviceIdType.MESH,
    )
    rdma.start()
    out_ref[...] = half_part(ix * T_LOC)
    rdma.wait()
    out_ref[...] += rbuf_ref[...].astype(jnp.float32)


def _rs_call(yg, sp):
    return pl.pallas_call(
        _rs_body,
        out_shape=jax.ShapeDtypeStruct((T_LOC, D), jnp.float32),
        in_specs=[
            pl.BlockSpec(memory_space=pltpu.VMEM),
            pl.BlockSpec(memory_space=pltpu.VMEM),
        ],
        out_specs=pl.BlockSpec(memory_space=pltpu.VMEM),
        scratch_shapes=[
            pltpu.VMEM((T_LOC, D), jnp.bfloat16),
            pltpu.VMEM((T_LOC, D), jnp.bfloat16),
            pltpu.SemaphoreType.DMA,
            pltpu.SemaphoreType.DMA,
        ],
        compiler_params=pltpu.CompilerParams(collective_id=1),
    )(yg, sp)


def kernel(x, router, W1, W2):
    ix = lax.axis_index("x")

    x_all, gates = _ag_call(x, router)

    tv, ti = lax.top_k(gates, K)
    w = jax.nn.softmax(tv, axis=1)
    flat_e = ti.reshape(-1)

    le = flat_e - E_LOC * ix
    local = (le >= 0) & (le < E_LOC)
    ar = jnp.arange(NPAIR, dtype=jnp.int32)

    key = jnp.where(local, le * NPAIR + ar, E_LOC * NPAIR + ar)
    order = jnp.argsort(key)
    j_of_p = jnp.argsort(order)

    counts = jnp.sum(
        (le[:, None] == jnp.arange(E_LOC)[None, :]) & local[:, None],
        axis=0,
        dtype=jnp.int32,
    )
    start = jnp.concatenate([jnp.zeros(1, jnp.int32), jnp.cumsum(counts)[:-1]])

    rank = j_of_p - start[jnp.clip(le, 0, E_LOC - 1)]
    slot = jnp.where(local & (rank < CAP), le * CAP + rank, NSLOT).astype(jnp.int32)

    sp = slot.reshape(T_GLB, K)
    s0 = sp[:, 0].reshape(1, T_GLB)
    s1 = sp[:, 1].reshape(1, T_GLB)
    w0 = w[:, 0].reshape(1, T_GLB)
    w1r = w[:, 1].reshape(1, T_GLB)

    yg = _ffn_call(x_all, s0, s1, w0, w1r, W1, W2)

    return _rs_call(yg.reshape(NSLOT, D), sp)


# device time: 178637 ns/iter; 1.3865x vs baseline; 1.0978x over previous
import jax
import jax.numpy as jnp
from jax import lax
from jax.experimental import pallas as pl
from jax.experimental.pallas import tpu as pltpu

T_LOC = 1024
T_GLB = 2048
D = 1024
F = 4096
E = 16
E_LOC = 8
K = 2
NPAIR = T_GLB * K
CAP = 320
NSLOT = E_LOC * CAP
FB = 1024


def _peer():
    ix = lax.axis_index("x")
    iy = lax.axis_index("y")
    iz = lax.axis_index("z")
    return ix, (1 - ix, iy, iz)


def _xpeer_barrier(peer):
    barrier = pltpu.get_barrier_semaphore()
    pl.semaphore_signal(
        barrier, inc=1, device_id=peer, device_id_type=pl.DeviceIdType.MESH
    )
    pl.semaphore_wait(barrier, 1)


def _ag_body(x_ref, rme_ref, xall_ref, gall_ref, rpeer_ref, send_sems, recv_sems):
    ix, peer = _peer()
    _xpeer_barrier(peer)

    my_off = ix * T_LOC

    r_rdma = pltpu.make_async_remote_copy(
        src_ref=rme_ref,
        dst_ref=rpeer_ref,
        send_sem=send_sems.at[0],
        recv_sem=recv_sems.at[0],
        device_id=peer,
        device_id_type=pl.DeviceIdType.MESH,
    )
    r_rdma.start()

    xall_ref[pl.ds(my_off, T_LOC), :] = x_ref[...].astype(jnp.bfloat16)
    x_rdma = pltpu.make_async_remote_copy(
        src_ref=xall_ref.at[pl.ds(my_off, T_LOC), :],
        dst_ref=xall_ref.at[pl.ds(my_off, T_LOC), :],
        send_sem=send_sems.at[1],
        recv_sem=recv_sems.at[1],
        device_id=peer,
        device_id_type=pl.DeviceIdType.MESH,
    )
    x_rdma.start()

    gme = jnp.dot(
        x_ref[...],
        rme_ref[...],
        preferred_element_type=jnp.float32,
        precision=lax.Precision.HIGHEST,
    )
    r_rdma.wait_recv()
    gpe = jnp.dot(
        x_ref[...],
        rpeer_ref[...],
        preferred_element_type=jnp.float32,
        precision=lax.Precision.HIGHEST,
    )
    cols = jnp.where(
        ix == 0,
        jnp.concatenate([gme, gpe], axis=1),
        jnp.concatenate([gpe, gme], axis=1),
    )
    gall_ref[pl.ds(my_off, T_LOC), :] = cols
    g_rdma = pltpu.make_async_remote_copy(
        src_ref=gall_ref.at[pl.ds(my_off, T_LOC), :],
        dst_ref=gall_ref.at[pl.ds(my_off, T_LOC), :],
        send_sem=send_sems.at[2],
        recv_sem=recv_sems.at[2],
        device_id=peer,
        device_id_type=pl.DeviceIdType.MESH,
    )
    g_rdma.start()

    r_rdma.wait_send()
    x_rdma.wait()
    g_rdma.wait()


def _ag_call(x, router):
    return pl.pallas_call(
        _ag_body,
        out_shape=[
            jax.ShapeDtypeStruct((T_GLB, D), jnp.bfloat16),
            jax.ShapeDtypeStruct((T_GLB, E), jnp.float32),
        ],
        in_specs=[
            pl.BlockSpec(memory_space=pltpu.VMEM),
            pl.BlockSpec(memory_space=pltpu.VMEM),
        ],
        out_specs=[
            pl.BlockSpec(memory_space=pltpu.VMEM),
            pl.BlockSpec(memory_space=pltpu.VMEM),
        ],
        scratch_shapes=[
            pltpu.VMEM((T_LOC, E_LOC), jnp.float32),
            pltpu.SemaphoreType.DMA((3,)),
            pltpu.SemaphoreType.DMA((3,)),
        ],
        compiler_params=pltpu.CompilerParams(collective_id=0),
    )(x, router)


def _ffn_body(
    xall_ref, s0_ref, s1_ref, w0_ref, w1r_ref, w1_ref, w2_ref, out_ref, xg_ref, acc_ref
):
    e = pl.program_id(0)
    f = pl.program_id(1)
    nf = pl.num_programs(1)

    @pl.when(f == 0)
    def _():
        slot_id = lax.broadcasted_iota(jnp.int32, (CAP, T_GLB), 0) + e * CAP
        m0 = slot_id == s0_ref[...]
        m1 = slot_id == s1_ref[...]
        ohw = jnp.where(m0, w0_ref[...], 0.0) + jnp.where(m1, w1r_ref[...], 0.0)
        xg_ref[...] = jnp.dot(
            ohw.astype(jnp.bfloat16),
            xall_ref[...],
            preferred_element_type=jnp.float32,
        ).astype(jnp.bfloat16)

    h = jnp.maximum(
        jnp.dot(
            xg_ref[...],
            w1_ref[0].astype(jnp.bfloat16),
            preferred_element_type=jnp.float32,
        ),
        0.0,
    )
    y = jnp.dot(
        h.astype(jnp.bfloat16),
        w2_ref[0].astype(jnp.bfloat16),
        preferred_element_type=jnp.float32,
    )

    @pl.when(f == 0)
    def _():
        acc_ref[...] = y

    @pl.when(f != 0)
    def _():
        acc_ref[...] += y

    @pl.when(f == nf - 1)
    def _():
        out_ref[0] = acc_ref[...].astype(jnp.bfloat16)


def _ffn_call(x_all, s0, s1, w0, w1r, W1, W2):
    return pl.pallas_call(
        _ffn_body,
        grid=(E_LOC, F // FB),
        out_shape=jax.ShapeDtypeStruct((E_LOC, CAP, D), jnp.bfloat16),
        in_specs=[
            pl.BlockSpec((T_GLB, D), lambda e, f: (0, 0)),
            pl.BlockSpec((1, T_GLB), lambda e, f: (0, 0)),
            pl.BlockSpec((1, T_GLB), lambda e, f: (0, 0)),
            pl.BlockSpec((1, T_GLB), lambda e, f: (0, 0)),
            pl.BlockSpec((1, T_GLB), lambda e, f: (0, 0)),
            pl.BlockSpec((1, D, FB), lambda e, f: (e, 0, f)),
            pl.BlockSpec((1, FB, D), lambda e, f: (e, f, 0)),
        ],
        out_specs=pl.BlockSpec((1, CAP, D), lambda e, f: (e, 0, 0)),
        scratch_shapes=[
            pltpu.VMEM((CAP, D), jnp.bfloat16),
            pltpu.VMEM((CAP, D), jnp.float32),
        ],
        compiler_params=pltpu.CompilerParams(
            dimension_semantics=("parallel", "arbitrary"),
        ),
    )(x_all, s0, s1, w0, w1r, W1, W2)


RS_NCH = 4
RS_CH = T_LOC // RS_NCH


def _rs_body(yg_ref, sp_ref, out_ref, sbuf_ref, rbuf_ref, send_sems, recv_sems):
    ix, peer = _peer()
    _xpeer_barrier(peer)

    def part_rows(off, n):
        s0 = sp_ref[pl.ds(off, n), 0:1]
        s1 = sp_ref[pl.ds(off, n), 1:2]
        slot_id = lax.broadcasted_iota(jnp.int32, (n, NSLOT), 1)
        oh = ((slot_id == s0) | (slot_id == s1)).astype(jnp.bfloat16)
        return jnp.dot(oh, yg_ref[...], preferred_element_type=jnp.float32)

    rdmas = []
    for c in range(RS_NCH):
        sbuf_ref[pl.ds(c * RS_CH, RS_CH), :] = part_rows(
            (1 - ix) * T_LOC + c * RS_CH, RS_CH
        ).astype(jnp.bfloat16)
        rdma = pltpu.make_async_remote_copy(
            src_ref=sbuf_ref.at[pl.ds(c * RS_CH, RS_CH), :],
            dst_ref=rbuf_ref.at[pl.ds(c * RS_CH, RS_CH), :],
            send_sem=send_sems.at[c],
            recv_sem=recv_sems.at[c],
            device_id=peer,
            device_id_type=pl.DeviceIdType.MESH,
        )
        rdma.start()
        rdmas.append(rdma)
    out_ref[...] = part_rows(ix * T_LOC, T_LOC)
    for rdma in rdmas:
        rdma.wait()
    out_ref[...] += rbuf_ref[...].astype(jnp.float32)


def _rs_call(yg, sp):
    return pl.pallas_call(
        _rs_body,
        out_shape=jax.ShapeDtypeStruct((T_LOC, D), jnp.float32),
        in_specs=[
            pl.BlockSpec(memory_space=pltpu.VMEM),
            pl.BlockSpec(memory_space=pltpu.VMEM),
        ],
        out_specs=pl.BlockSpec(memory_space=pltpu.VMEM),
        scratch_shapes=[
            pltpu.VMEM((T_LOC, D), jnp.bfloat16),
            pltpu.VMEM((T_LOC, D), jnp.bfloat16),
            pltpu.SemaphoreType.DMA((RS_NCH,)),
            pltpu.SemaphoreType.DMA((RS_NCH,)),
        ],
        compiler_params=pltpu.CompilerParams(collective_id=1),
    )(yg, sp)


def kernel(x, router, W1, W2):
    ix = lax.axis_index("x")

    x_all, gates = _ag_call(x, router)

    tv, ti = lax.top_k(gates, K)
    w = jax.nn.softmax(tv, axis=1)
    flat_e = ti.reshape(-1)

    le = flat_e - E_LOC * ix
    local = (le >= 0) & (le < E_LOC)

    oh = (le[:, None] == jnp.arange(E_LOC)[None, :]) & local[:, None]
    pos = jnp.cumsum(oh.astype(jnp.int32), axis=0)
    rank = jnp.sum(jnp.where(oh, pos - 1, 0), axis=1)
    slot = jnp.where(local & (rank < CAP), le * CAP + rank, NSLOT).astype(jnp.int32)

    sp = slot.reshape(T_GLB, K)
    s0 = sp[:, 0].reshape(1, T_GLB)
    s1 = sp[:, 1].reshape(1, T_GLB)
    w0 = w[:, 0].reshape(1, T_GLB)
    w1r = w[:, 1].reshape(1, T_GLB)

    yg = _ffn_call(x_all, s0, s1, w0, w1r, W1, W2)

    return _rs_call(yg.reshape(NSLOT, D), sp)
